# Initial kernel scaffold; baseline (speedup 1.0000x reference)
import jax
import jax.numpy as jnp
from jax.experimental import pallas as pl


def kernel(preds, targs, label_lengths):
    raise NotImplementedError("write your pallas kernel here")


# speedup vs baseline: 2.8148x; 2.8148x over previous
import jax
import jax.numpy as jnp
from jax.experimental import pallas as pl

_CH = 256


def _ssl_body(preds_ref, targs_ref, out_ref):
    b = pl.program_id(0)
    n = preds_ref.shape[2]
    px = preds_ref[0, 0:1, :]
    py = preds_ref[0, 1:2, :]

    def body(c, ones):
        j0 = c * _CH
        tx = targs_ref[0, pl.ds(j0, _CH), 0:1]
        ty = targs_ref[0, pl.ds(j0, _CH), 1:2]
        tm = targs_ref[0, pl.ds(j0, _CH), 2:3]
        dx = tx - px
        dy = ty - py
        d2 = dx * dx + dy * dy
        m = jnp.min(d2, axis=1, keepdims=True)
        hit = jnp.logical_and(d2 <= m, tm != 0.0)
        onesc = jnp.max(hit.astype(jnp.float32), axis=0, keepdims=True)
        return jnp.maximum(ones, onesc)

    ones = jax.lax.fori_loop(0, n // _CH, body, jnp.zeros((1, n), jnp.float32))

    p = preds_ref[0, 2:3, :]
    p = jnp.clip(p, 1e-12, 1.0 - 1e-12)
    term = ones * jnp.log(p) + (1.0 - ones) * jnp.log(1.0 - p)
    bce = -jnp.sum(term) / n

    @pl.when(b == 0)
    def _():
        out_ref[...] = jnp.zeros_like(out_ref)

    out_ref[...] += bce


def kernel(preds, targs, label_lengths):
    del label_lengths
    B, N, _ = preds.shape
    preds_t = jnp.transpose(preds, (0, 2, 1))
    out = pl.pallas_call(
        _ssl_body,
        grid=(B,),
        in_specs=[
            pl.BlockSpec((1, 3, N), lambda b: (b, 0, 0)),
            pl.BlockSpec((1, N, 3), lambda b: (b, 0, 0)),
        ],
        out_specs=pl.BlockSpec((1, 1), lambda b: (0, 0)),
        out_shape=jax.ShapeDtypeStruct((1, 1), jnp.float32),
    )(preds_t, targs)
    return out[0, 0]


# speedup vs baseline: 3.5580x; 1.2641x over previous
import jax
import jax.numpy as jnp
from jax.experimental import pallas as pl

_CH = 256


def _ssl_body(preds_ref, targs_ref, out_ref):
    b = pl.program_id(0)
    n = preds_ref.shape[2]
    px = preds_ref[0, 0:1, :]
    py = preds_ref[0, 1:2, :]

    def body(c, acc):
        j0 = c * _CH
        tx = targs_ref[0, pl.ds(j0, _CH), 0:1]
        ty = targs_ref[0, pl.ds(j0, _CH), 1:2]
        tm = targs_ref[0, pl.ds(j0, _CH), 2:3]
        dx = tx - px
        dy = ty - py
        d2 = dx * dx + dy * dy
        m = jnp.min(d2, axis=1, keepdims=True)
        m = jnp.where(tm != 0.0, m, -jnp.inf)
        e = jnp.min(d2 - m, axis=0, keepdims=True)
        return jnp.minimum(acc, e)

    acc = jax.lax.fori_loop(
        0, n // _CH, body, jnp.full((1, n), jnp.inf, jnp.float32)
    )
    ones = jnp.where(acc <= 0.0, 1.0, 0.0)

    p = preds_ref[0, 2:3, :]
    p = jnp.clip(p, 1e-12, 1.0 - 1e-12)
    term = ones * jnp.log(p) + (1.0 - ones) * jnp.log(1.0 - p)
    bce = -jnp.sum(term) / n

    @pl.when(b == 0)
    def _():
        out_ref[...] = jnp.zeros_like(out_ref)

    out_ref[...] += bce


def kernel(preds, targs, label_lengths):
    del label_lengths
    B, N, _ = preds.shape
    preds_t = jnp.transpose(preds, (0, 2, 1))
    out = pl.pallas_call(
        _ssl_body,
        grid=(B,),
        in_specs=[
            pl.BlockSpec((1, 3, N), lambda b: (b, 0, 0)),
            pl.BlockSpec((1, N, 3), lambda b: (b, 0, 0)),
        ],
        out_specs=pl.BlockSpec((1, 1), lambda b: (0, 0)),
        out_shape=jax.ShapeDtypeStruct((1, 1), jnp.float32),
    )(preds_t, targs)
    return out[0, 0]
